# baseline (speedup 1.0000x reference)
import functools

import jax
import jax.numpy as jnp
from jax import lax
from jax.experimental import pallas as pl
from jax.experimental.pallas import tpu as pltpu
from jax.experimental.pallas import tpu_sc as plsc

N_TABLES = 26
VOCAB = 100000
DIM = 64
BATCH = 4096
L = N_TABLES * BATCH

NC = 2
NS = 16
LANES = 16
NW = NC * NS
B_PER_W = L // NW
CHUNK = 416
N_CHUNKS = B_PER_W // CHUNK
LOG2_BATCH = 12


def _sc_gather(pair_w, flat_idx):
    mesh = plsc.VectorSubcoreMesh(core_axis_name="c", subcore_axis_name="s")

    @functools.partial(
        pl.kernel,
        mesh=mesh,
        out_type=jax.ShapeDtypeStruct((L, DIM), jnp.float32),
        scratch_types=[
            pltpu.VMEM((CHUNK,), jnp.int32),
            pltpu.VMEM((CHUNK,), jnp.int32),
            pltpu.VMEM((CHUNK, 2 * DIM), jnp.float32),
            pltpu.SemaphoreType.DMA,
        ],
        compiler_params=pltpu.CompilerParams(
            use_tc_tiling_on_sc=False, needs_layout_passes=False
        ),
    )
    def k(w_hbm, idx_hbm, out_hbm, idx_v, half_v, rows_v, sem):
        wid = lax.axis_index("s") * NC + lax.axis_index("c")
        base = wid * B_PER_W

        def chunk_body(c, _):
            cbase = base + c * CHUNK
            pltpu.sync_copy(idx_hbm.at[pl.ds(cbase, CHUNK)], idx_v)

            def adj(j, _):
                row = cbase + j * LANES + lax.iota(jnp.int32, 16)
                tid = lax.shift_right_logical(row, LOG2_BATCH)
                off = j * LANES
                flat = idx_v[pl.ds(off, LANES)] + tid * VOCAB
                idx_v[pl.ds(off, LANES)] = lax.shift_right_logical(flat, 1)
                half_v[pl.ds(off, LANES)] = (flat & 1) * DIM
                return 0

            lax.fori_loop(0, CHUNK // LANES, adj, 0)
            pltpu.async_copy(w_hbm.at[idx_v], rows_v, sem).wait()

            p16 = lax.iota(jnp.int32, 16)

            def sel_g(g, _):
                r16 = g * LANES + p16
                hv = half_v[pl.ds(g * LANES, LANES)]
                m = hv > 0
                for col in range(DIM):
                    val = plsc.load_gather(rows_v, [r16, hv + col], mask=m)
                    plsc.store_scatter(
                        rows_v,
                        [r16, jnp.full((LANES,), col, jnp.int32)],
                        val,
                        mask=m,
                    )
                return 0

            lax.fori_loop(0, CHUNK // LANES, sel_g, 0)
            pltpu.sync_copy(
                rows_v.at[:, pl.ds(0, DIM)], out_hbm.at[pl.ds(cbase, CHUNK)]
            )
            return 0

        lax.fori_loop(0, N_CHUNKS, chunk_body, 0)

    return k(pair_w, flat_idx)


def kernel(indices, offsets, W):
    del offsets
    pair_w = W.reshape(N_TABLES * VOCAB // 2, 2 * DIM)
    flat_idx = indices.astype(jnp.int32)
    out = _sc_gather(pair_w, flat_idx)
    return out.reshape(N_TABLES, BATCH, DIM)


# speedup vs baseline: 4.3391x; 4.3391x over previous
import functools

import jax
import jax.numpy as jnp
from jax import lax
from jax.experimental import pallas as pl
from jax.experimental.pallas import tpu as pltpu
from jax.experimental.pallas import tpu_sc as plsc

N_TABLES = 26
VOCAB = 100000
DIM = 64
BATCH = 4096
L = N_TABLES * BATCH

NC = 2
NS = 16
LANES = 16
NW = NC * NS
UNITS = N_TABLES * DIM
U_PER_W = UNITS // NW
LOG2_DIM = 6


def _sc_lookup(wt, idx):
    mesh = plsc.VectorSubcoreMesh(core_axis_name="c", subcore_axis_name="s")

    @functools.partial(
        pl.kernel,
        mesh=mesh,
        out_type=jax.ShapeDtypeStruct((N_TABLES, DIM, BATCH), jnp.float32),
        scratch_types=[
            pltpu.VMEM((VOCAB,), jnp.float32),
            pltpu.VMEM((BATCH,), jnp.int32),
            pltpu.VMEM((BATCH,), jnp.float32),
            pltpu.SemaphoreType.DMA,
        ],
        compiler_params=pltpu.CompilerParams(needs_layout_passes=False),
    )
    def k(wt_hbm, idx_hbm, out_hbm, row_v, idx_v, out_v, sem):
        wid = lax.axis_index("s") * NC + lax.axis_index("c")
        u0 = wid * U_PER_W

        def unit_body(u, _):
            t = lax.shift_right_logical(u, LOG2_DIM)
            d = u & (DIM - 1)
            pltpu.sync_copy(idx_hbm.at[pl.ds(t * BATCH, BATCH)], idx_v)
            pltpu.sync_copy(wt_hbm.at[t, d, :], row_v)

            def g(j, _):
                v16 = idx_v[pl.ds(j * LANES, LANES)]
                out_v[pl.ds(j * LANES, LANES)] = plsc.load_gather(row_v, [v16])
                return 0

            lax.fori_loop(0, BATCH // LANES, g, 0)
            pltpu.sync_copy(out_v, out_hbm.at[t, d, :])
            return 0

        lax.fori_loop(u0, u0 + U_PER_W, unit_body, 0)

    return k(wt, idx)


def kernel(indices, offsets, W):
    del offsets
    wt = jnp.transpose(W, (0, 2, 1))
    flat_idx = indices.astype(jnp.int32)
    out_t = _sc_lookup(wt, flat_idx)
    return jnp.transpose(out_t, (0, 2, 1))


# speedup vs baseline: 4.6798x; 1.0785x over previous
import functools

import jax
import jax.numpy as jnp
from jax import lax
from jax.experimental import pallas as pl
from jax.experimental.pallas import tpu as pltpu
from jax.experimental.pallas import tpu_sc as plsc

N_TABLES = 26
VOCAB = 100000
DIM = 64
BATCH = 4096
L = N_TABLES * BATCH

NC = 2
NS = 16
LANES = 16
NW = NC * NS
UNITS = N_TABLES * DIM
U_PER_W = UNITS // NW
LOG2_DIM = 6

SPLIT = 49920
BLEN = 50048
TAIL = 128
BHI = SPLIT + BLEN
ABUF = SPLIT + TAIL


def _sc_lookup(wt, tail_wt, idx):
    mesh = plsc.VectorSubcoreMesh(core_axis_name="c", subcore_axis_name="s")

    @functools.partial(
        pl.kernel,
        mesh=mesh,
        out_type=jax.ShapeDtypeStruct((N_TABLES, DIM, BATCH), jnp.float32),
        scratch_types=[
            pltpu.VMEM((ABUF,), jnp.float32),
            pltpu.VMEM((BLEN,), jnp.float32),
            pltpu.VMEM((BATCH,), jnp.int32),
            pltpu.VMEM((BATCH,), jnp.float32),
            pltpu.SemaphoreType.DMA,
            pltpu.SemaphoreType.DMA,
        ],
        compiler_params=pltpu.CompilerParams(needs_layout_passes=False),
    )
    def k(wt_hbm, tail_hbm, idx_hbm, out_hbm, a_v, b_v, idx_v, out_v, sa, sb):
        wid = lax.axis_index("s") * NC + lax.axis_index("c")
        u0 = wid * U_PER_W
        u_end = u0 + U_PER_W
        p16 = lax.iota(jnp.int32, 16)

        def td(u):
            return lax.shift_right_logical(u, LOG2_DIM), u & (DIM - 1)

        def start_a(u):
            t, d = td(u)
            pltpu.async_copy(
                wt_hbm.at[t, d, pl.ds(0, SPLIT)], a_v.at[pl.ds(0, SPLIT)], sa)
            pltpu.async_copy(
                tail_hbm.at[t, d, :], a_v.at[pl.ds(SPLIT, TAIL)], sa)

        def start_b(u):
            t, d = td(u)
            pltpu.async_copy(
                wt_hbm.at[t, d, pl.ds(SPLIT, BLEN)], b_v, sb)

        t0, _ = td(u0)
        pltpu.sync_copy(idx_hbm.at[pl.ds(t0 * BATCH, BATCH)], idx_v)
        start_a(u0)
        start_b(u0)

        def unit_body(u, _):
            t, d = td(u)

            @pl.when(jnp.logical_and(u != u0, d == 0))
            def _():
                pltpu.sync_copy(idx_hbm.at[pl.ds(t * BATCH, BATCH)], idx_v)

            pltpu.make_async_copy(
                wt_hbm.at[0, 0, pl.ds(0, SPLIT)], a_v.at[pl.ds(0, SPLIT)], sa
            ).wait()
            pltpu.make_async_copy(
                tail_hbm.at[0, 0, :], a_v.at[pl.ds(SPLIT, TAIL)], sa
            ).wait()

            def ga(j, _):
                v = idx_v[pl.ds(j * LANES, LANES)]
                m1 = v < SPLIT
                m = jnp.logical_or(m1, v >= BHI)
                va = jnp.where(m1, v, v - (VOCAB - TAIL) + SPLIT)
                va = jnp.where(m, va, 0)
                g = plsc.load_gather(a_v, [va], mask=m)
                out_v[pl.ds(j * LANES, LANES)] = jnp.where(m, g, 0.0)
                return 0

            lax.fori_loop(0, BATCH // LANES, ga, 0)

            @pl.when(u + 1 < u_end)
            def _():
                start_a(u + 1)

            pltpu.make_async_copy(
                wt_hbm.at[0, 0, pl.ds(SPLIT, BLEN)], b_v, sb
            ).wait()

            def gb(j, _):
                v = idx_v[pl.ds(j * LANES, LANES)]
                m = jnp.logical_and(v >= SPLIT, v < BHI)
                vc = jnp.where(m, v - SPLIT, 0)
                g = plsc.load_gather(b_v, [vc], mask=m)
                plsc.store_scatter(out_v, [j * LANES + p16], g, mask=m)
                return 0

            lax.fori_loop(0, BATCH // LANES, gb, 0)

            @pl.when(u + 1 < u_end)
            def _():
                start_b(u + 1)

            pltpu.sync_copy(out_v, out_hbm.at[t, d, :])
            return 0

        lax.fori_loop(u0, u_end, unit_body, 0)

    return k(wt, tail_wt, idx)


def kernel(indices, offsets, W):
    del offsets
    wt = jnp.transpose(W, (0, 2, 1))
    tail_wt = wt[:, :, VOCAB - TAIL:]
    flat_idx = indices.astype(jnp.int32)
    out_t = _sc_lookup(wt, tail_wt, flat_idx)
    return jnp.transpose(out_t, (0, 2, 1))
